# baseline (speedup 1.0000x reference)
import functools

import jax
import jax.numpy as jnp
from jax import lax
from jax.experimental import pallas as pl
from jax.experimental.pallas import tpu as pltpu
from jax.experimental.pallas import tpu_sc as plsc


def _sc_gather_chunk(table, idx, chunk_base, nrows):
    D = table.shape[1]
    info = plsc.get_sparse_core_info()
    nc, ns = info.num_cores, info.num_subcores
    nw = nc * ns
    b_per_w = nrows // nw
    mesh = plsc.VectorSubcoreMesh(core_axis_name="c", subcore_axis_name="s")

    @functools.partial(
        pl.kernel,
        mesh=mesh,
        out_type=jax.ShapeDtypeStruct((nrows, D), jnp.float32),
        scratch_types=[
            pltpu.VMEM((b_per_w,), jnp.int32),
            pltpu.VMEM((b_per_w, D), jnp.float32),
            pltpu.SemaphoreType.DMA,
        ],
    )
    def gather_kernel(table_hbm, idx_hbm, out_hbm, idx_v, rows_v, sem):
        wid = lax.axis_index("s") * nc + lax.axis_index("c")
        base = wid * b_per_w
        pltpu.sync_copy(idx_hbm.at[pl.ds(chunk_base + base, b_per_w)], idx_v)
        pltpu.async_copy(table_hbm.at[idx_v], rows_v, sem).wait()
        pltpu.sync_copy(rows_v, out_hbm.at[pl.ds(base, b_per_w)])

    return gather_kernel(table, idx)


def _add_body(x_ref, s_ref, o_ref):
    o_ref[...] = x_ref[...] + s_ref[...][:, None, :]


def _add_body_aliased(prev_ref, x_ref, s_ref, o_ref):
    del prev_ref
    o_ref[...] = x_ref[...] + s_ref[...][:, None, :]


def kernel(x, subject_idx, embedding_table):
    B, F, T = x.shape
    idx = subject_idx.astype(jnp.int32)

    B0 = 512
    B1 = B - B0
    subj0 = _sc_gather_chunk(embedding_table, idx, 0, B0)
    subj1 = _sc_gather_chunk(embedding_table, idx, B0, B1)

    xt = x.transpose(0, 2, 1)

    bb = 128
    partial = pl.pallas_call(
        _add_body,
        grid=(B0 // bb,),
        in_specs=[
            pl.BlockSpec((bb, T, F), lambda i: (i, 0, 0)),
            pl.BlockSpec((bb, F), lambda i: (i, 0)),
        ],
        out_specs=pl.BlockSpec((bb, T, F), lambda i: (i, 0, 0)),
        out_shape=jax.ShapeDtypeStruct((B, T, F), jnp.float32),
        compiler_params=pltpu.CompilerParams(
            dimension_semantics=("parallel",)
        ),
    )(xt, subj0)

    nskip = B0 // bb
    outt = pl.pallas_call(
        _add_body_aliased,
        grid=(B1 // bb,),
        in_specs=[
            pl.BlockSpec(memory_space=pl.ANY),
            pl.BlockSpec((bb, T, F), lambda i: (i + nskip, 0, 0)),
            pl.BlockSpec((bb, F), lambda i: (i, 0)),
        ],
        out_specs=pl.BlockSpec((bb, T, F), lambda i: (i + nskip, 0, 0)),
        out_shape=jax.ShapeDtypeStruct((B, T, F), jnp.float32),
        input_output_aliases={0: 0},
        compiler_params=pltpu.CompilerParams(
            dimension_semantics=("parallel",)
        ),
    )(partial, xt, subj1)
    return outt.transpose(0, 2, 1)


# speedup vs baseline: 1.0006x; 1.0006x over previous
import functools

import jax
import jax.numpy as jnp
from jax import lax
from jax.experimental import pallas as pl
from jax.experimental.pallas import tpu as pltpu
from jax.experimental.pallas import tpu_sc as plsc


def _sc_gather_chunk(table, idx, chunk_base, nrows):
    D = table.shape[1]
    info = plsc.get_sparse_core_info()
    nc, ns = info.num_cores, info.num_subcores
    nw = nc * ns
    b_per_w = nrows // nw
    mesh = plsc.VectorSubcoreMesh(core_axis_name="c", subcore_axis_name="s")

    @functools.partial(
        pl.kernel,
        mesh=mesh,
        out_type=jax.ShapeDtypeStruct((nrows, D), jnp.float32),
        scratch_types=[
            pltpu.VMEM((b_per_w,), jnp.int32),
            pltpu.VMEM((b_per_w, D), jnp.float32),
            pltpu.SemaphoreType.DMA,
        ],
    )
    def gather_kernel(table_hbm, idx_hbm, out_hbm, idx_v, rows_v, sem):
        wid = lax.axis_index("s") * nc + lax.axis_index("c")
        base = wid * b_per_w
        pltpu.sync_copy(idx_hbm.at[pl.ds(chunk_base + base, b_per_w)], idx_v)
        pltpu.async_copy(table_hbm.at[idx_v], rows_v, sem).wait()
        pltpu.sync_copy(rows_v, out_hbm.at[pl.ds(base, b_per_w)])

    return gather_kernel(table, idx)


def _add_body(x_ref, s_ref, o_ref):
    o_ref[...] = x_ref[...] + s_ref[...][:, None, :]


def _add_body_aliased(prev_ref, x_ref, s_ref, o_ref):
    del prev_ref
    o_ref[...] = x_ref[...] + s_ref[...][:, None, :]


def kernel(x, subject_idx, embedding_table):
    B, F, T = x.shape
    idx = subject_idx.astype(jnp.int32)

    B0 = 256
    B1 = B - B0
    subj0 = _sc_gather_chunk(embedding_table, idx, 0, B0)
    subj1 = _sc_gather_chunk(embedding_table, idx, B0, B1)

    xt = x.transpose(0, 2, 1)

    bb = 128
    partial = pl.pallas_call(
        _add_body,
        grid=(B0 // bb,),
        in_specs=[
            pl.BlockSpec((bb, T, F), lambda i: (i, 0, 0)),
            pl.BlockSpec((bb, F), lambda i: (i, 0)),
        ],
        out_specs=pl.BlockSpec((bb, T, F), lambda i: (i, 0, 0)),
        out_shape=jax.ShapeDtypeStruct((B, T, F), jnp.float32),
        compiler_params=pltpu.CompilerParams(
            dimension_semantics=("parallel",)
        ),
    )(xt, subj0)

    nskip = B0 // bb
    outt = pl.pallas_call(
        _add_body_aliased,
        grid=(B1 // bb,),
        in_specs=[
            pl.BlockSpec(memory_space=pl.ANY),
            pl.BlockSpec((bb, T, F), lambda i: (i + nskip, 0, 0)),
            pl.BlockSpec((bb, F), lambda i: (i, 0)),
        ],
        out_specs=pl.BlockSpec((bb, T, F), lambda i: (i + nskip, 0, 0)),
        out_shape=jax.ShapeDtypeStruct((B, T, F), jnp.float32),
        input_output_aliases={0: 0},
        compiler_params=pltpu.CompilerParams(
            dimension_semantics=("parallel",)
        ),
    )(partial, xt, subj1)
    return outt.transpose(0, 2, 1)
